# baseline (speedup 1.0000x reference)
import functools
import math as _math

import jax
import jax.numpy as jnp
from jax import lax
from jax.experimental import pallas as pl
from jax.experimental.pallas import tpu as pltpu
from jax.experimental.pallas import tpu_sc as plsc

_V = 32000
_N = 4096
_SMOOTHING = 0.1
_CONF = 1.0 - _SMOOTHING
_EPS = _SMOOTHING / (_V - 2)
_C = _SMOOTHING * _math.log(_EPS) + _CONF * _math.log(_CONF)

_NC = 2
_NS = 16
_NW = _NC * _NS
_LANES = 16
_RPW = _N // _NW
_GROUPS = _RPW // _LANES
_VREGS = _V // _LANES
_UNROLL = 40
_OUTER = _VREGS // _UNROLL


def _row_sum_vec(buf_ref):
    zero = jnp.zeros((_LANES,), jnp.float32)

    def body(j, accs):
        accs = list(accs)
        off = j * (_UNROLL * _LANES)
        for k in range(_UNROLL):
            v = buf_ref[pl.ds(off + k * _LANES, _LANES)]
            accs[k % 4] = accs[k % 4] + v
        return tuple(accs)

    a0, a1, a2, a3 = lax.fori_loop(0, _OUTER, body, (zero, zero, zero, zero))
    return (a0 + a1) + (a2 + a3)


def _take16(v, idx):
    return lax.gather(
        v, idx[:, None],
        dimension_numbers=lax.GatherDimensionNumbers(
            offset_dims=(), collapsed_slice_dims=(0,), start_index_map=(0,)
        ),
        slice_sizes=(1,),
        mode=lax.GatherScatterMode.PROMISE_IN_BOUNDS,
    )


def _allsum_bc(v, lane):
    for sh in (8, 4, 2, 1):
        v = v + _take16(v, lane ^ sh)
    return v


@functools.cache
def _build_sc_loss():
    mesh = plsc.VectorSubcoreMesh(
        core_axis_name="c", subcore_axis_name="s", num_cores=_NC
    )

    @functools.partial(
        pl.kernel,
        mesh=mesh,
        out_type=jax.ShapeDtypeStruct((_NW, _LANES), jnp.float32),
        scratch_types=[
            pltpu.VMEM((_RPW,), jnp.int32),
            pltpu.VMEM((_RPW,), jnp.int32),
            pltpu.VMEM((_RPW,), jnp.float32),
            pltpu.VMEM((_V,), jnp.float32),
            pltpu.VMEM((_V,), jnp.float32),
            pltpu.VMEM((_LANES,), jnp.float32),
            pltpu.SemaphoreType.DMA,
            pltpu.SemaphoreType.DMA,
            pltpu.SemaphoreType.DMA,
        ],
    )
    def _sc_loss(lp2d_hbm, lp_flat_hbm, tgt_hbm, out_hbm,
                 tgt_v, idx_v, gat_v, buf0, buf1, acc_v,
                 sem_g, sem0, sem1):
        wid = lax.axis_index("s") * _NC + lax.axis_index("c")
        base = wid * _RPW

        pltpu.async_copy(lp2d_hbm.at[base], buf0, sem0)
        pltpu.async_copy(lp2d_hbm.at[base + 1], buf1, sem1)

        pltpu.sync_copy(tgt_hbm.at[pl.ds(base, _RPW)], tgt_v)

        lane = lax.iota(jnp.int32, _LANES)
        for c in range(_GROUPS):
            t16 = tgt_v[pl.ds(c * _LANES, _LANES)]
            row = (base + c * _LANES) + lane
            idx_v[pl.ds(c * _LANES, _LANES)] = row * _V + t16

        pltpu.async_copy(lp_flat_hbm.at[idx_v], gat_v, sem_g).wait()

        zero16 = jnp.zeros((_LANES,), jnp.float32)

        def group_body(cg, acc):
            grp = zero16
            for l in range(_LANES):
                buf, sem = (buf0, sem0) if l % 2 == 0 else (buf1, sem1)
                pltpu.make_async_copy(lp2d_hbm.at[base], buf, sem).wait()
                rsb = _allsum_bc(_row_sum_vec(buf), lane)
                lp0b = _take16(buf[pl.ds(0, _LANES)], lane & 0)
                nxt = cg * _LANES + l + 2

                @pl.when(nxt < _RPW)
                def _():
                    pltpu.async_copy(lp2d_hbm.at[base + nxt], buf, sem)

                grp = jnp.where(lane == l, rsb - lp0b, grp)

            o = cg * _LANES
            t16 = tgt_v[pl.ds(o, _LANES)]
            g16 = gat_v[pl.ds(o, _LANES)]
            c16 = _C - _EPS * grp - (_CONF - _EPS) * g16
            return acc + jnp.where(t16 != 0, c16, jnp.float32(0.0))

        acc = lax.fori_loop(0, _GROUPS, group_body, zero16)
        acc_v[...] = acc
        pltpu.sync_copy(acc_v, out_hbm.at[wid])

    return _sc_loss


def kernel(log_probs, target):
    tgt = target.astype(jnp.int32)
    partials = _build_sc_loss()(log_probs, log_probs.reshape(-1), tgt)
    return jnp.sum(partials)


# speedup vs baseline: 1.0100x; 1.0100x over previous
import functools
import math as _math

import jax
import jax.numpy as jnp
from jax import lax
from jax.experimental import pallas as pl
from jax.experimental.pallas import tpu as pltpu
from jax.experimental.pallas import tpu_sc as plsc

_V = 32000
_N = 4096
_SMOOTHING = 0.1
_CONF = 1.0 - _SMOOTHING
_EPS = _SMOOTHING / (_V - 2)
_C = _SMOOTHING * _math.log(_EPS) + _CONF * _math.log(_CONF)

_NC = 2
_NS = 16
_NW = _NC * _NS
_LANES = 16
_RPW = _N // _NW
_GROUPS = _RPW // _LANES
_VREGS = _V // _LANES
_UNROLL = 40
_OUTER = _VREGS // _UNROLL


def _row_sum_vec(buf_ref):
    zero = jnp.zeros((_LANES,), jnp.float32)

    def body(j, accs):
        accs = list(accs)
        off = j * (_UNROLL * _LANES)
        for k in range(_UNROLL):
            v = buf_ref[pl.ds(off + k * _LANES, _LANES)]
            accs[k % 4] = accs[k % 4] + v
        return tuple(accs)

    a0, a1, a2, a3 = lax.fori_loop(0, _OUTER, body, (zero, zero, zero, zero))
    return (a0 + a1) + (a2 + a3)


def _take16(v, idx):
    return lax.gather(
        v, idx[:, None],
        dimension_numbers=lax.GatherDimensionNumbers(
            offset_dims=(), collapsed_slice_dims=(0,), start_index_map=(0,)
        ),
        slice_sizes=(1,),
        mode=lax.GatherScatterMode.PROMISE_IN_BOUNDS,
    )


def _allsum_bc(v, lane):
    for sh in (8, 4, 2, 1):
        v = v + _take16(v, lane ^ sh)
    return v


@functools.cache
def _build_sc_loss():
    mesh = plsc.VectorSubcoreMesh(
        core_axis_name="c", subcore_axis_name="s", num_cores=_NC
    )

    @functools.partial(
        pl.kernel,
        mesh=mesh,
        out_type=jax.ShapeDtypeStruct((_NW, _LANES), jnp.float32),
        scratch_types=[
            pltpu.VMEM((_RPW,), jnp.int32),
            pltpu.VMEM((_RPW,), jnp.int32),
            pltpu.VMEM((_RPW,), jnp.float32),
            pltpu.VMEM((_V,), jnp.float32),
            pltpu.VMEM((_V,), jnp.float32),
            pltpu.VMEM((_LANES,), jnp.float32),
            pltpu.SemaphoreType.DMA,
            pltpu.SemaphoreType.DMA,
            pltpu.SemaphoreType.DMA,
        ],
    )
    def _sc_loss(lp_flat_hbm, tgt_hbm, out_hbm,
                 tgt_v, idx_v, gat_v, buf0, buf1, acc_v,
                 sem_g, sem0, sem1):
        wid = lax.axis_index("s") * _NC + lax.axis_index("c")
        base = wid * _RPW

        def row_src(r):
            return lp_flat_hbm.at[pl.ds(r * _V, _V)]

        pltpu.async_copy(row_src(base), buf0, sem0)
        pltpu.async_copy(row_src(base + 1), buf1, sem1)

        pltpu.sync_copy(tgt_hbm.at[pl.ds(base, _RPW)], tgt_v)

        lane = lax.iota(jnp.int32, _LANES)
        for c in range(_GROUPS):
            t16 = tgt_v[pl.ds(c * _LANES, _LANES)]
            row = (base + c * _LANES) + lane
            idx_v[pl.ds(c * _LANES, _LANES)] = row * _V + t16

        pltpu.async_copy(lp_flat_hbm.at[idx_v], gat_v, sem_g).wait()

        zero16 = jnp.zeros((_LANES,), jnp.float32)

        def group_body(cg, acc):
            grp = zero16
            for l in range(_LANES):
                buf, sem = (buf0, sem0) if l % 2 == 0 else (buf1, sem1)
                pltpu.make_async_copy(row_src(base), buf, sem).wait()
                rsb = _allsum_bc(_row_sum_vec(buf), lane)
                lp0b = _take16(buf[pl.ds(0, _LANES)], lane & 0)
                nxt = cg * _LANES + l + 2

                @pl.when(nxt < _RPW)
                def _():
                    pltpu.async_copy(row_src(base + nxt), buf, sem)

                grp = jnp.where(lane == l, rsb - lp0b, grp)

            o = cg * _LANES
            t16 = tgt_v[pl.ds(o, _LANES)]
            g16 = gat_v[pl.ds(o, _LANES)]
            c16 = _C - _EPS * grp - (_CONF - _EPS) * g16
            return acc + jnp.where(t16 != 0, c16, jnp.float32(0.0))

        acc = lax.fori_loop(0, _GROUPS, group_body, zero16)
        acc_v[...] = acc
        pltpu.sync_copy(acc_v, out_hbm.at[wid])

    return _sc_loss


def kernel(log_probs, target):
    tgt = target.astype(jnp.int32)
    partials = _build_sc_loss()(log_probs.reshape(-1), tgt)
    return jnp.sum(partials)


# speedup vs baseline: 1.8783x; 1.8598x over previous
import functools
import math as _math

import jax
import jax.numpy as jnp
from jax import lax
from jax.experimental import pallas as pl
from jax.experimental.pallas import tpu as pltpu
from jax.experimental.pallas import tpu_sc as plsc

_V = 32000
_N = 4096
_SMOOTHING = 0.1
_CONF = 1.0 - _SMOOTHING
_EPS = _SMOOTHING / (_V - 2)
_C = _SMOOTHING * _math.log(_EPS) + _CONF * _math.log(_CONF)

_NC = 2
_NS = 16
_NW = _NC * _NS
_LANES = 16
_RPW = _N // _NW
_GROUPS = _RPW // _LANES
_VREGS = _V // _LANES
_UNROLL = 40
_OUTER = _VREGS // _UNROLL


def _row_sum_vec(buf_ref, t_bc, lane):
    zero = jnp.zeros((_LANES,), jnp.float32)

    def body(j, carry):
        a0, a1, a2, a3, g = carry
        accs = [a0, a1, a2, a3]
        off = j * (_UNROLL * _LANES)
        ebase = lane + off
        for k in range(_UNROLL):
            v = buf_ref[pl.ds(off + k * _LANES, _LANES)]
            accs[k % 4] = accs[k % 4] + v
            g = jnp.where(ebase == t_bc - (k * _LANES), v, g)
        return (*accs, g)

    a0, a1, a2, a3, g = lax.fori_loop(
        0, _OUTER, body, (zero, zero, zero, zero, zero)
    )
    return (a0 + a1) + (a2 + a3), g


def _take16(v, idx):
    return lax.gather(
        v, idx[:, None],
        dimension_numbers=lax.GatherDimensionNumbers(
            offset_dims=(), collapsed_slice_dims=(0,), start_index_map=(0,)
        ),
        slice_sizes=(1,),
        mode=lax.GatherScatterMode.PROMISE_IN_BOUNDS,
    )


def _allsum_bc(v, lane):
    for sh in (8, 4, 2, 1):
        v = v + _take16(v, lane ^ sh)
    return v


@functools.cache
def _build_sc_loss():
    mesh = plsc.VectorSubcoreMesh(
        core_axis_name="c", subcore_axis_name="s", num_cores=_NC
    )

    @functools.partial(
        pl.kernel,
        mesh=mesh,
        out_type=jax.ShapeDtypeStruct((_NW, _LANES), jnp.float32),
        scratch_types=[
            pltpu.VMEM((_RPW,), jnp.int32),
            pltpu.VMEM((_V,), jnp.float32),
            pltpu.VMEM((_V,), jnp.float32),
            pltpu.VMEM((_LANES,), jnp.float32),
            pltpu.SemaphoreType.DMA,
            pltpu.SemaphoreType.DMA,
        ],
    )
    def _sc_loss(lp_hbm, tgt_hbm, out_hbm,
                 tgt_v, buf0, buf1, acc_v, sem0, sem1):
        wid = lax.axis_index("s") * _NC + lax.axis_index("c")
        base = wid * _RPW

        pltpu.async_copy(lp_hbm.at[base], buf0, sem0)
        pltpu.async_copy(lp_hbm.at[base + 1], buf1, sem1)

        pltpu.sync_copy(tgt_hbm.at[pl.ds(base, _RPW)], tgt_v)

        lane = lax.iota(jnp.int32, _LANES)
        zero16 = jnp.zeros((_LANES,), jnp.float32)

        def group_body(cg, acc):
            t16 = tgt_v[pl.ds(cg * _LANES, _LANES)]
            grp = zero16
            ggrp = zero16
            for l in range(_LANES):
                buf, sem = (buf0, sem0) if l % 2 == 0 else (buf1, sem1)
                pltpu.make_async_copy(lp_hbm.at[base], buf, sem).wait()
                t_bc = _take16(t16, (lane & 0) + l)
                rs, g = _row_sum_vec(buf, t_bc, lane)
                rsb = _allsum_bc(rs, lane)
                g_bc = _allsum_bc(g, lane)
                lp0b = _take16(buf[pl.ds(0, _LANES)], lane & 0)
                nxt = cg * _LANES + l + 2

                @pl.when(nxt < _RPW)
                def _():
                    pltpu.async_copy(lp_hbm.at[base + nxt], buf, sem)

                grp = jnp.where(lane == l, rsb - lp0b, grp)
                ggrp = jnp.where(lane == l, g_bc, ggrp)

            c16 = _C - _EPS * grp - (_CONF - _EPS) * ggrp
            return acc + jnp.where(t16 != 0, c16, jnp.float32(0.0))

        acc = lax.fori_loop(0, _GROUPS, group_body, zero16)
        acc_v[...] = acc
        pltpu.sync_copy(acc_v, out_hbm.at[wid])

    return _sc_loss


def kernel(log_probs, target):
    tgt = target.astype(jnp.int32)
    partials = _build_sc_loss()(log_probs, tgt)
    return jnp.sum(partials)


# speedup vs baseline: 2.7579x; 1.4683x over previous
import functools
import math as _math

import jax
import jax.numpy as jnp
from jax import lax
from jax.experimental import pallas as pl
from jax.experimental.pallas import tpu as pltpu
from jax.experimental.pallas import tpu_sc as plsc

_V = 32000
_N = 4096
_SMOOTHING = 0.1
_CONF = 1.0 - _SMOOTHING
_EPS = _SMOOTHING / (_V - 2)
_C = _SMOOTHING * _math.log(_EPS) + _CONF * _math.log(_CONF)

_NT = 1024
_RT = 128

_NC = 2
_NS = 16
_NW = _NC * _NS
_LANES = 16
_RPW = (_N - _NT) // _NW
_GROUPS = _RPW // _LANES
_VREGS = _V // _LANES
_UNROLL = 40
_OUTER = _VREGS // _UNROLL


def _tc_body(x_ref, t_ref, o_ref):
    i = pl.program_id(0)
    x = x_ref[...]
    t = t_ref[...]
    rowsum = jnp.sum(x, axis=1)
    lp0 = x[:, 0]
    colid = lax.broadcasted_iota(jnp.int32, (_RT, _V), 1)
    lpt = jnp.sum(jnp.where(colid == t[:, None], x, 0.0), axis=1)
    contrib = _C - _EPS * (rowsum - lp0) - (_CONF - _EPS) * lpt
    tot = jnp.sum(jnp.where(t != 0, contrib, 0.0))

    @pl.when(i == 0)
    def _():
        o_ref[0, 0] = tot

    @pl.when(i > 0)
    def _():
        o_ref[0, 0] = o_ref[0, 0] + tot


def _tc_partial(log_probs, target):
    return pl.pallas_call(
        _tc_body,
        grid=(_NT // _RT,),
        in_specs=[
            pl.BlockSpec((_RT, _V), lambda i: (i, 0)),
            pl.BlockSpec((_RT,), lambda i: (i,)),
        ],
        out_specs=pl.BlockSpec(memory_space=pltpu.SMEM),
        out_shape=jax.ShapeDtypeStruct((1, 1), jnp.float32),
        compiler_params=pltpu.CompilerParams(
            dimension_semantics=("arbitrary",)
        ),
    )(log_probs, target)


def _row_sum_vec(buf_ref, t_bc, lane):
    zero = jnp.zeros((_LANES,), jnp.float32)

    def body(j, carry):
        a0, a1, a2, a3, g = carry
        accs = [a0, a1, a2, a3]
        off = j * (_UNROLL * _LANES)
        tb = t_bc - (lane + off)
        for k in range(_UNROLL):
            v = buf_ref[pl.ds(off + k * _LANES, _LANES)]
            accs[k % 4] = accs[k % 4] + v
            g = jnp.where(tb == k * _LANES, v, g)
        return (*accs, g)

    a0, a1, a2, a3, g = lax.fori_loop(
        0, _OUTER, body, (zero, zero, zero, zero, zero)
    )
    return (a0 + a1) + (a2 + a3), g


def _take16(v, idx):
    return lax.gather(
        v, idx[:, None],
        dimension_numbers=lax.GatherDimensionNumbers(
            offset_dims=(), collapsed_slice_dims=(0,), start_index_map=(0,)
        ),
        slice_sizes=(1,),
        mode=lax.GatherScatterMode.PROMISE_IN_BOUNDS,
    )


def _allsum_bc(v, lane):
    for sh in (8, 4, 2, 1):
        v = v + _take16(v, lane ^ sh)
    return v


@functools.cache
def _build_sc_loss():
    mesh = plsc.VectorSubcoreMesh(
        core_axis_name="c", subcore_axis_name="s", num_cores=_NC
    )

    @functools.partial(
        pl.kernel,
        mesh=mesh,
        out_type=jax.ShapeDtypeStruct((_NW, _LANES), jnp.float32),
        scratch_types=[
            pltpu.VMEM((_RPW,), jnp.int32),
            pltpu.VMEM((_V,), jnp.float32),
            pltpu.VMEM((_V,), jnp.float32),
            pltpu.VMEM((_LANES,), jnp.float32),
            pltpu.SemaphoreType.DMA,
            pltpu.SemaphoreType.DMA,
        ],
    )
    def _sc_loss(lp_hbm, tgt_hbm, out_hbm,
                 tgt_v, buf0, buf1, acc_v, sem0, sem1):
        wid = lax.axis_index("s") * _NC + lax.axis_index("c")
        base = _NT + wid * _RPW

        pltpu.async_copy(lp_hbm.at[base], buf0, sem0)
        pltpu.async_copy(lp_hbm.at[base + 1], buf1, sem1)

        pltpu.sync_copy(tgt_hbm.at[pl.ds(base, _RPW)], tgt_v)

        lane = lax.iota(jnp.int32, _LANES)
        zero16 = jnp.zeros((_LANES,), jnp.float32)

        def group_body(cg, acc):
            t16 = tgt_v[pl.ds(cg * _LANES, _LANES)]
            grp = zero16
            ggrp = zero16
            for l in range(_LANES):
                buf, sem = (buf0, sem0) if l % 2 == 0 else (buf1, sem1)
                pltpu.make_async_copy(lp_hbm.at[base], buf, sem).wait()
                t_bc = _take16(t16, (lane & 0) + l)
                rs, g = _row_sum_vec(buf, t_bc, lane)
                rsb = _allsum_bc(rs, lane)
                g_bc = _allsum_bc(g, lane)
                lp0b = _take16(buf[pl.ds(0, _LANES)], lane & 0)
                nxt = cg * _LANES + l + 2

                @pl.when(nxt < _RPW)
                def _():
                    pltpu.async_copy(lp_hbm.at[base + nxt], buf, sem)

                grp = jnp.where(lane == l, rsb - lp0b, grp)
                ggrp = jnp.where(lane == l, g_bc, ggrp)

            c16 = _C - _EPS * grp - (_CONF - _EPS) * ggrp
            return acc + jnp.where(t16 != 0, c16, jnp.float32(0.0))

        acc = lax.fori_loop(0, _GROUPS, group_body, zero16)
        acc_v[...] = acc
        pltpu.sync_copy(acc_v, out_hbm.at[wid])

    return _sc_loss


def kernel(log_probs, target):
    tgt = target.astype(jnp.int32)
    tc_part = _tc_partial(log_probs, tgt)
    partials = _build_sc_loss()(log_probs, tgt)
    return tc_part[0, 0] + jnp.sum(partials)


# speedup vs baseline: 3.1065x; 1.1264x over previous
import functools
import math as _math

import jax
import jax.numpy as jnp
from jax import lax
from jax.experimental import pallas as pl
from jax.experimental.pallas import tpu as pltpu
from jax.experimental.pallas import tpu_sc as plsc

_V = 32000
_N = 4096
_SMOOTHING = 0.1
_CONF = 1.0 - _SMOOTHING
_EPS = _SMOOTHING / (_V - 2)
_C = _SMOOTHING * _math.log(_EPS) + _CONF * _math.log(_CONF)

_NT = 2048
_RT = 128

_NC = 2
_NS = 16
_NW = _NC * _NS
_LANES = 16
_RPW = (_N - _NT) // _NW
_GROUPS = _RPW // _LANES
_VREGS = _V // _LANES
_UNROLL = 40
_OUTER = _VREGS // _UNROLL


def _tc_body(x_ref, t_ref, o_ref):
    i = pl.program_id(0)
    x = x_ref[...]
    t = t_ref[...]
    rowsum = jnp.sum(x, axis=1)
    lp0 = x[:, 0]
    colid = lax.broadcasted_iota(jnp.int32, (_RT, _V), 1)
    lpt = jnp.sum(jnp.where(colid == t[:, None], x, 0.0), axis=1)
    contrib = _C - _EPS * (rowsum - lp0) - (_CONF - _EPS) * lpt
    tot = jnp.sum(jnp.where(t != 0, contrib, 0.0))

    @pl.when(i == 0)
    def _():
        o_ref[0, 0] = tot

    @pl.when(i > 0)
    def _():
        o_ref[0, 0] = o_ref[0, 0] + tot


def _tc_partial(log_probs, target):
    return pl.pallas_call(
        _tc_body,
        grid=(_NT // _RT,),
        in_specs=[
            pl.BlockSpec((_RT, _V), lambda i: (i, 0)),
            pl.BlockSpec((_RT,), lambda i: (i,)),
        ],
        out_specs=pl.BlockSpec(memory_space=pltpu.SMEM),
        out_shape=jax.ShapeDtypeStruct((1, 1), jnp.float32),
        compiler_params=pltpu.CompilerParams(
            dimension_semantics=("arbitrary",)
        ),
    )(log_probs, target)


def _row_sum_vec(buf_ref, t_bc, lane):
    zero = jnp.zeros((_LANES,), jnp.float32)

    def body(j, carry):
        a0, a1, a2, a3, g = carry
        accs = [a0, a1, a2, a3]
        off = j * (_UNROLL * _LANES)
        tb = t_bc - (lane + off)
        for k in range(_UNROLL):
            v = buf_ref[pl.ds(off + k * _LANES, _LANES)]
            accs[k % 4] = accs[k % 4] + v
            g = jnp.where(tb == k * _LANES, v, g)
        return (*accs, g)

    a0, a1, a2, a3, g = lax.fori_loop(
        0, _OUTER, body, (zero, zero, zero, zero, zero)
    )
    return (a0 + a1) + (a2 + a3), g


def _take16(v, idx):
    return lax.gather(
        v, idx[:, None],
        dimension_numbers=lax.GatherDimensionNumbers(
            offset_dims=(), collapsed_slice_dims=(0,), start_index_map=(0,)
        ),
        slice_sizes=(1,),
        mode=lax.GatherScatterMode.PROMISE_IN_BOUNDS,
    )


def _allsum_bc(v, lane):
    for sh in (8, 4, 2, 1):
        v = v + _take16(v, lane ^ sh)
    return v


@functools.cache
def _build_sc_loss():
    mesh = plsc.VectorSubcoreMesh(
        core_axis_name="c", subcore_axis_name="s", num_cores=_NC
    )

    @functools.partial(
        pl.kernel,
        mesh=mesh,
        out_type=jax.ShapeDtypeStruct((_NW, _LANES), jnp.float32),
        scratch_types=[
            pltpu.VMEM((_RPW,), jnp.int32),
            pltpu.VMEM((_V,), jnp.float32),
            pltpu.VMEM((_V,), jnp.float32),
            pltpu.VMEM((_V,), jnp.float32),
            pltpu.VMEM((_V,), jnp.float32),
            pltpu.VMEM((_LANES,), jnp.float32),
            pltpu.SemaphoreType.DMA,
            pltpu.SemaphoreType.DMA,
            pltpu.SemaphoreType.DMA,
            pltpu.SemaphoreType.DMA,
        ],
    )
    def _sc_loss(lp_hbm, tgt_hbm, out_hbm,
                 tgt_v, buf0, buf1, buf2, buf3, acc_v,
                 sem0, sem1, sem2, sem3):
        wid = lax.axis_index("s") * _NC + lax.axis_index("c")
        base = _NT + wid * _RPW
        bufs = (buf0, buf1, buf2, buf3)
        sems = (sem0, sem1, sem2, sem3)

        for b in range(4):
            pltpu.async_copy(lp_hbm.at[base + b], bufs[b], sems[b])

        pltpu.sync_copy(tgt_hbm.at[pl.ds(base, _RPW)], tgt_v)

        lane = lax.iota(jnp.int32, _LANES)
        zero16 = jnp.zeros((_LANES,), jnp.float32)

        def group_body(cg, acc):
            t16 = tgt_v[pl.ds(cg * _LANES, _LANES)]
            grp = zero16
            ggrp = zero16
            for l in range(_LANES):
                buf, sem = bufs[l % 4], sems[l % 4]
                pltpu.make_async_copy(lp_hbm.at[base], buf, sem).wait()
                t_bc = _take16(t16, (lane & 0) + l)
                rs, g = _row_sum_vec(buf, t_bc, lane)
                rsb = _allsum_bc(rs, lane)
                g_bc = _allsum_bc(g, lane)
                lp0b = _take16(buf[pl.ds(0, _LANES)], lane & 0)
                nxt = cg * _LANES + l + 4

                @pl.when(nxt < _RPW)
                def _():
                    pltpu.async_copy(lp_hbm.at[base + nxt], buf, sem)

                grp = jnp.where(lane == l, rsb - lp0b, grp)
                ggrp = jnp.where(lane == l, g_bc, ggrp)

            c16 = _C - _EPS * grp - (_CONF - _EPS) * ggrp
            return acc + jnp.where(t16 != 0, c16, jnp.float32(0.0))

        acc = lax.fori_loop(0, _GROUPS, group_body, zero16)
        acc_v[...] = acc
        pltpu.sync_copy(acc_v, out_hbm.at[wid])

    return _sc_loss


def kernel(log_probs, target):
    tgt = target.astype(jnp.int32)
    tc_part = _tc_partial(log_probs, tgt)
    partials = _build_sc_loss()(log_probs, tgt)
    return tc_part[0, 0] + jnp.sum(partials)


# speedup vs baseline: 3.1510x; 1.0143x over previous
import functools
import math as _math

import jax
import jax.numpy as jnp
from jax import lax
from jax.experimental import pallas as pl
from jax.experimental.pallas import tpu as pltpu
from jax.experimental.pallas import tpu_sc as plsc

_V = 32000
_N = 4096
_SMOOTHING = 0.1
_CONF = 1.0 - _SMOOTHING
_EPS = _SMOOTHING / (_V - 2)
_C = _SMOOTHING * _math.log(_EPS) + _CONF * _math.log(_CONF)

_NT = 2048
_RT = 128

_NC = 2
_NS = 16
_NW = _NC * _NS
_LANES = 16
_RPW = (_N - _NT) // _NW
_GROUPS = _RPW // _LANES
_VREGS = _V // _LANES
_UNROLL = 40
_OUTER = _VREGS // _UNROLL


def _tc_body(x_ref, t_ref, o_ref):
    i = pl.program_id(0)
    x = x_ref[...]
    t = t_ref[...]
    rowsum = jnp.sum(x, axis=1)
    lp0 = x[:, 0]
    colid = lax.broadcasted_iota(jnp.int32, (_RT, _V), 1)
    lpt = jnp.sum(jnp.where(colid == t[:, None], x, 0.0), axis=1)
    contrib = _C - _EPS * (rowsum - lp0) - (_CONF - _EPS) * lpt
    tot = jnp.sum(jnp.where(t != 0, contrib, 0.0))

    @pl.when(i == 0)
    def _():
        o_ref[0, 0] = tot

    @pl.when(i > 0)
    def _():
        o_ref[0, 0] = o_ref[0, 0] + tot


def _tc_partial(log_probs, target):
    return pl.pallas_call(
        _tc_body,
        grid=(_NT // _RT,),
        in_specs=[
            pl.BlockSpec((_RT, _V), lambda i: (i, 0)),
            pl.BlockSpec((_RT,), lambda i: (i,)),
        ],
        out_specs=pl.BlockSpec(memory_space=pltpu.SMEM),
        out_shape=jax.ShapeDtypeStruct((1, 1), jnp.float32),
        compiler_params=pltpu.CompilerParams(
            dimension_semantics=("arbitrary",)
        ),
    )(log_probs, target)


def _row_sum_vec(buf_ref, t_bc, lane):
    zero = jnp.zeros((_LANES,), jnp.float32)

    def body(j, carry):
        a0, a1, a2, a3, g = carry
        accs = [a0, a1, a2, a3]
        off = j * (_UNROLL * _LANES)
        tb = t_bc - (lane + off)
        for k in range(_UNROLL):
            v = buf_ref[pl.ds(off + k * _LANES, _LANES)]
            accs[k % 4] = accs[k % 4] + v
            g = jnp.where(tb == k * _LANES, v, g)
        return (*accs, g)

    a0, a1, a2, a3, g = lax.fori_loop(
        0, _OUTER, body, (zero, zero, zero, zero, zero)
    )
    return (a0 + a1) + (a2 + a3), g


def _take16(v, idx):
    return lax.gather(
        v, idx[:, None],
        dimension_numbers=lax.GatherDimensionNumbers(
            offset_dims=(), collapsed_slice_dims=(0,), start_index_map=(0,)
        ),
        slice_sizes=(1,),
        mode=lax.GatherScatterMode.PROMISE_IN_BOUNDS,
    )


def _allsum_bc(v, lane):
    for sh in (8, 4, 2, 1):
        v = v + _take16(v, lane ^ sh)
    return v


@functools.cache
def _build_sc_loss():
    mesh = plsc.VectorSubcoreMesh(
        core_axis_name="c", subcore_axis_name="s", num_cores=_NC
    )

    @functools.partial(
        pl.kernel,
        mesh=mesh,
        out_type=jax.ShapeDtypeStruct((_NW, _LANES), jnp.float32),
        scratch_types=[
            pltpu.VMEM((_RPW,), jnp.int32),
            pltpu.VMEM((_V,), jnp.float32),
            pltpu.VMEM((_V,), jnp.float32),
            pltpu.VMEM((_LANES,), jnp.float32),
            pltpu.SemaphoreType.DMA,
            pltpu.SemaphoreType.DMA,
        ],
    )
    def _sc_loss(lp_hbm, tgt_hbm, out_hbm,
                 tgt_v, buf0, buf1, acc_v, sem0, sem1):
        wid = lax.axis_index("s") * _NC + lax.axis_index("c")
        base = _NT + wid * _RPW

        pltpu.async_copy(lp_hbm.at[base], buf0, sem0)
        pltpu.async_copy(lp_hbm.at[base + 1], buf1, sem1)

        pltpu.sync_copy(tgt_hbm.at[pl.ds(base, _RPW)], tgt_v)

        lane = lax.iota(jnp.int32, _LANES)
        zero16 = jnp.zeros((_LANES,), jnp.float32)

        def group_body(cg, acc):
            t16 = tgt_v[pl.ds(cg * _LANES, _LANES)]
            grp = zero16
            ggrp = zero16
            for l in range(_LANES):
                buf, sem = (buf0, sem0) if l % 2 == 0 else (buf1, sem1)
                pltpu.make_async_copy(lp_hbm.at[base], buf, sem).wait()
                t_bc = _take16(t16, (lane & 0) + l)
                rs, g = _row_sum_vec(buf, t_bc, lane)
                rsb = _allsum_bc(rs, lane)
                g_bc = _allsum_bc(g, lane)
                lp0b = _take16(buf[pl.ds(0, _LANES)], lane & 0)
                nxt = cg * _LANES + l + 2

                @pl.when(nxt < _RPW)
                def _():
                    pltpu.async_copy(lp_hbm.at[base + nxt], buf, sem)

                grp = jnp.where(lane == l, rsb - lp0b, grp)
                ggrp = jnp.where(lane == l, g_bc, ggrp)

            c16 = _C - _EPS * grp - (_CONF - _EPS) * ggrp
            return acc + jnp.where(t16 != 0, c16, jnp.float32(0.0))

        acc = lax.fori_loop(0, _GROUPS, group_body, zero16)
        acc_v[...] = acc
        pltpu.sync_copy(acc_v, out_hbm.at[wid])

    return _sc_loss


def kernel(log_probs, target):
    tgt = target.astype(jnp.int32)
    tc_part = _tc_partial(log_probs, tgt)
    partials = _build_sc_loss()(log_probs, tgt)
    return tc_part[0, 0] + jnp.sum(partials)
